# Initial kernel scaffold; baseline (speedup 1.0000x reference)
import jax
import jax.numpy as jnp
from jax.experimental import pallas as pl


def kernel(k):
    raise NotImplementedError("write your pallas kernel here")


# speedup vs baseline: 9.9993x; 9.9993x over previous
import functools

import jax
import jax.numpy as jnp
import numpy as np
from jax.experimental import pallas as pl
from jax.experimental.pallas import tpu as pltpu


def _dft_mats(n: int):
    i = np.arange(n)
    m = np.outer(i, i) % n
    th = (2.0 * np.pi / n) * m
    return np.cos(th).astype(np.float32), np.sin(th).astype(np.float32)


def _body(c_ref, s_ref, x_ref, o_ref, *, mh, n):
    f32 = jnp.float32
    x = x_ref[0]
    c = c_ref[...]
    s = s_ref[...]
    ch = c[:mh, :]
    sh = s[:mh, :]
    a = jnp.dot(ch, x, preferred_element_type=f32)
    b = jnp.dot(sh, x, preferred_element_type=f32)
    ab = a.astype(jnp.bfloat16)
    bb = b.astype(jnp.bfloat16)
    zr = (jnp.dot(ab, c, preferred_element_type=f32)
          - jnp.dot(bb, s, preferred_element_type=f32))
    zi = (jnp.dot(ab, s, preferred_element_type=f32)
          + jnp.dot(bb, c, preferred_element_type=f32))
    mag = jnp.sqrt(zr * zr + zi * zi)
    u = jax.lax.broadcasted_iota(jnp.int32, (mh, n), 0)
    w = jnp.where((u == 0) | (u == n // 2), 1.0,
                  jnp.where(u < n // 2, 2.0, 0.0)).astype(f32)
    total = jnp.sum(mag * w)
    o_ref[...] = jnp.full((1, 8, 128), total, dtype=f32)


def _inv_loss(x, interpret=False):
    nb, n, _ = x.shape
    mh = n // 2 + 8
    cnp, snp = _dft_mats(n)
    cb = jnp.asarray(cnp, dtype=jnp.bfloat16)
    sb = jnp.asarray(snp, dtype=jnp.bfloat16)
    xb = x.astype(jnp.bfloat16)
    per_image = pl.pallas_call(
        functools.partial(_body, mh=mh, n=n),
        grid=(nb,),
        in_specs=[
            pl.BlockSpec((n, n), lambda i: (0, 0)),
            pl.BlockSpec((n, n), lambda i: (0, 0)),
            pl.BlockSpec((1, n, n), lambda i: (i, 0, 0)),
        ],
        out_specs=pl.BlockSpec((1, 8, 128), lambda i: (i, 0, 0)),
        out_shape=jax.ShapeDtypeStruct((nb, 8, 128), jnp.float32),
        compiler_params=pltpu.CompilerParams(
            dimension_semantics=("parallel",),
        ),
        name="inv_loss_fft_mag",
        interpret=interpret,
    )(cb, sb, xb)
    return jnp.mean(per_image[:, 0, 0])


def kernel(k):
    bsz, ch, h, w = k.shape
    return _inv_loss(k.reshape(bsz * ch, h, w))
